# Initial kernel scaffold; baseline (speedup 1.0000x reference)
import jax
import jax.numpy as jnp
from jax.experimental import pallas as pl


def kernel(nodes, senders, receivers, W, b):
    raise NotImplementedError("write your pallas kernel here")


# speedup vs baseline: 4.5236x; 4.5236x over previous
import functools

import jax
import jax.numpy as jnp
from jax import lax
from jax.experimental import pallas as pl
from jax.experimental.pallas import tpu as pltpu
from jax.experimental.pallas import tpu_sc as plsc

_CH = 80


def _sc_info():
    info = plsc.get_sparse_core_info()
    return info.num_cores, info.num_subcores


def _stripes(n, ns):
    stripe = (n // (ns * 8)) * 8
    tail = n - ns * stripe
    zr = stripe // 3
    assert stripe % 3 == 0 and tail % 8 == 0 and tail <= zr
    return stripe, tail, zr


def _sc_segment_sum(nodes, senders, receivers):
    n, d = nodes.shape
    e = senders.shape[0]
    nc, ns = _sc_info()
    nw = nc * ns
    ew = e // nw
    nchunks = ew // _CH
    stripe, tail, zr = _stripes(n, ns)

    mesh = plsc.VectorSubcoreMesh(core_axis_name="c", subcore_axis_name="s")

    @functools.partial(
        pl.kernel,
        mesh=mesh,
        out_type=jax.ShapeDtypeStruct((nc, n, d), jnp.float32),
        scratch_types=[
            pltpu.VMEM((_CH,), jnp.int32),
            pltpu.VMEM((_CH,), jnp.int32),
            pltpu.VMEM((_CH, d), jnp.float32),
            pltpu.VMEM((zr, d), jnp.float32),
            pltpu.VMEM_SHARED((n, d), jnp.float32),
            pltpu.SemaphoreType.DMA,
        ],
    )
    def sc_kernel(nodes_hbm, send_hbm, recv_hbm, sums_out,
                  idx_s, idx_r, rows, zrow, acc, sem):
        c = lax.axis_index("c")
        s = lax.axis_index("s")
        wid = s * nc + c

        def fill_row(i, _):
            for j in range(d // 16):
                zrow[i, pl.ds(j * 16, 16)] = jnp.zeros((16,), jnp.float32)
            return 0

        lax.fori_loop(0, zr, fill_row, 0)

        base_row = s * stripe
        for r in range(3):
            pltpu.sync_copy(zrow, acc.at[pl.ds(base_row + r * zr, zr)])

        @pl.when(s == 0)
        def _zero_tail():
            pltpu.sync_copy(zrow.at[pl.ds(0, tail)],
                            acc.at[pl.ds(ns * stripe, tail)])

        plsc.subcore_barrier()

        ebase = wid * ew

        def chunk_body(i, _):
            off = ebase + i * _CH
            pltpu.sync_copy(send_hbm.at[pl.ds(off, _CH)], idx_s)
            pltpu.sync_copy(recv_hbm.at[pl.ds(off, _CH)], idx_r)
            pltpu.async_copy(nodes_hbm.at[idx_s], rows, sem).wait()
            pltpu.sync_copy(rows, acc.at[idx_r], add=True)
            return 0

        lax.fori_loop(0, nchunks, chunk_body, 0)

        plsc.subcore_barrier()

        pltpu.sync_copy(acc.at[pl.ds(base_row, stripe)],
                        sums_out.at[c, pl.ds(base_row, stripe)])

        @pl.when(s == 0)
        def _copy_tail():
            pltpu.sync_copy(acc.at[pl.ds(ns * stripe, tail)],
                            sums_out.at[c, pl.ds(ns * stripe, tail)])

    return sc_kernel(nodes, senders, receivers)


def _sc_segment_count(receivers, n, d):
    e = receivers.shape[0]
    nc, ns = _sc_info()
    nw = nc * ns
    ew = e // nw
    nchunks = ew // _CH
    stripe, tail, zr = _stripes(n, ns)

    mesh = plsc.VectorSubcoreMesh(core_axis_name="c", subcore_axis_name="s")

    @functools.partial(
        pl.kernel,
        mesh=mesh,
        out_type=jax.ShapeDtypeStruct((nc, n, d), jnp.float32),
        scratch_types=[
            pltpu.VMEM((_CH,), jnp.int32),
            pltpu.VMEM((_CH, d), jnp.float32),
            pltpu.VMEM((zr, d), jnp.float32),
            pltpu.VMEM_SHARED((n, d), jnp.float32),
        ],
    )
    def sc_kernel(recv_hbm, cnts_out, idx_r, ones, zcnt, cnt):
        c = lax.axis_index("c")
        s = lax.axis_index("s")
        wid = s * nc + c

        def fill_row(i, _):
            for j in range(d // 16):
                zcnt[i, pl.ds(j * 16, 16)] = jnp.zeros((16,), jnp.float32)
            return 0

        lax.fori_loop(0, zr, fill_row, 0)

        def fill_ones(i, _):
            for j in range(d // 16):
                ones[i, pl.ds(j * 16, 16)] = jnp.ones((16,), jnp.float32)
            return 0

        lax.fori_loop(0, _CH, fill_ones, 0)

        base_row = s * stripe
        for r in range(3):
            pltpu.sync_copy(zcnt, cnt.at[pl.ds(base_row + r * zr, zr)])

        @pl.when(s == 0)
        def _zero_tail():
            pltpu.sync_copy(zcnt.at[pl.ds(0, tail)],
                            cnt.at[pl.ds(ns * stripe, tail)])

        plsc.subcore_barrier()

        ebase = wid * ew

        def chunk_body(i, _):
            off = ebase + i * _CH
            pltpu.sync_copy(recv_hbm.at[pl.ds(off, _CH)], idx_r)
            pltpu.sync_copy(ones, cnt.at[idx_r], add=True)
            return 0

        lax.fori_loop(0, nchunks, chunk_body, 0)

        plsc.subcore_barrier()

        pltpu.sync_copy(cnt.at[pl.ds(base_row, stripe)],
                        cnts_out.at[c, pl.ds(base_row, stripe)])

        @pl.when(s == 0)
        def _copy_tail():
            pltpu.sync_copy(cnt.at[pl.ds(ns * stripe, tail)],
                            cnts_out.at[c, pl.ds(ns * stripe, tail)])

    return sc_kernel(receivers)


def _tc_finish(sums, cnts, nodes, w1, w2, b2):
    n, d = nodes.shape
    h = w1.shape[1]
    nc = sums.shape[0]
    rows = 2000
    grid = (n // rows,)

    def tc_kernel(sums_ref, cnts_ref, nodes_ref, w1_ref, w2_ref, b_ref, out_ref):
        ssum = sums_ref[0]
        csum = cnts_ref[0, :, 0:1]
        for k in range(1, nc):
            ssum = ssum + sums_ref[k]
            csum = csum + cnts_ref[k, :, 0:1]
        he = ssum / jnp.maximum(csum, 1.0)
        acc = jnp.dot(he, w1_ref[...], preferred_element_type=jnp.float32)
        acc = acc + jnp.dot(nodes_ref[...], w2_ref[...],
                            preferred_element_type=jnp.float32)
        out_ref[...] = jnp.maximum(acc + b_ref[...], 0.0)

    return pl.pallas_call(
        tc_kernel,
        grid=grid,
        in_specs=[
            pl.BlockSpec((nc, rows, d), lambda i: (0, i, 0)),
            pl.BlockSpec((nc, rows, d), lambda i: (0, i, 0)),
            pl.BlockSpec((rows, d), lambda i: (i, 0)),
            pl.BlockSpec((d, h), lambda i: (0, 0)),
            pl.BlockSpec((d, h), lambda i: (0, 0)),
            pl.BlockSpec((1, h), lambda i: (0, 0)),
        ],
        out_specs=pl.BlockSpec((rows, h), lambda i: (i, 0)),
        out_shape=jax.ShapeDtypeStruct((n, h), jnp.float32),
    )(sums, cnts, nodes, w1, w2, b2)


def kernel(nodes, senders, receivers, W, b):
    n, d = nodes.shape
    senders = senders.astype(jnp.int32)
    receivers = receivers.astype(jnp.int32)
    sums = _sc_segment_sum(nodes, senders, receivers)
    cnts = _sc_segment_count(receivers, n, d)
    return _tc_finish(sums, cnts, nodes, W[:d], W[d:], b.reshape(1, -1))
